# Initial kernel scaffold; baseline (speedup 1.0000x reference)
import jax
import jax.numpy as jnp
from jax.experimental import pallas as pl


def kernel(node_features, edge_features, action_mask, edge_index, ln_n_g, ln_n_b, ln_e_g, ln_e_b, W0, We0, as0, ad0, ae0, b0, W1, We1, as1, ad1, ae1, b1, W2, We2, as2, ad2, ae2, b2, W_out, b_out):
    raise NotImplementedError("write your pallas kernel here")


# speedup vs baseline: 4.0821x; 4.0821x over previous
import jax
import jax.numpy as jnp
from jax.experimental import pallas as pl
from jax.experimental.pallas import tpu as pltpu

_B, _N, _E = 8, 2048, 32768
_EB = 256
_NB = _E // _EB
_F32 = jnp.float32

_T_LHS = (((0,), (0,)), ((), ()))


def _lrelu(x):
    return jnp.where(x >= 0, x, 0.2 * x)


def _elu(x):
    return jnp.where(x > 0, x, jnp.exp(jnp.minimum(x, 0.0)) - 1.0)


def _gat_fwd_kernel(nf_ref, ef_ref, ei_ref,
                    ln_ng_ref, ln_nb_ref, ln_eg_ref, ln_eb_ref,
                    W0_ref, We0_ref, as0_ref, ad0_ref, ae0_ref, b0_ref,
                    W1_ref, We1_ref, as1_ref, ad1_ref, ae1_ref, b1_ref,
                    W2_ref, We2_ref, as2_ref, ad2_ref, ae2_ref, b2_ref,
                    Wout_ref, bout_ref,
                    out_ref,
                    ea_ref, hin_ref, hproj_ref, acc_ref):
    f32 = _F32

    x = nf_ref[0]
    mu = jnp.mean(x, axis=-1, keepdims=True)
    var = jnp.mean((x - mu) ** 2, axis=-1, keepdims=True)
    x = (x - mu) * jax.lax.rsqrt(var + 1e-5) * ln_ng_ref[0:1, :] + ln_nb_ref[0:1, :]

    g_e = ln_eg_ref[:, 0:1]
    b_e = ln_eb_ref[:, 0:1]

    def ln_blk(i, carry):
        sl = pl.ds(i * _EB, _EB)
        ea = ef_ref[0, :, sl]
        mu_e = jnp.mean(ea, axis=0, keepdims=True)
        var_e = jnp.mean((ea - mu_e) ** 2, axis=0, keepdims=True)
        ea_ref[:, sl] = (ea - mu_e) * jax.lax.rsqrt(var_e + 1e-5) * g_e + b_e
        return carry

    jax.lax.fori_loop(0, _NB, ln_blk, 0)

    layers = (
        (W0_ref, We0_ref, as0_ref, ad0_ref, ae0_ref, b0_ref),
        (W1_ref, We1_ref, as1_ref, ad1_ref, ae1_ref, b1_ref),
        (W2_ref, We2_ref, as2_ref, ad2_ref, ae2_ref, b2_ref),
    )

    hfin = None
    for li, (W_r, We_r, as_r, ad_r, ae_r, b_r) in enumerate(layers):
        W = W_r[...]
        We = We_r[...]
        a_s = as_r[0:1, :]
        a_d = ad_r[0:1, :]
        a_e = ae_r[0:1, :]

        h_prev = x if li == 0 else hin_ref[...]
        hproj_ref[...] = jnp.dot(h_prev, W, preferred_element_type=f32)
        h = hproj_ref[...]

        s_d = jnp.sum(h * a_d, axis=1, keepdims=True)
        s_dw = jnp.broadcast_to(s_d, (_N, 128))
        w_ae = jnp.sum(We * a_e, axis=1, keepdims=True)

        def ee_blk(i, m):
            sl = pl.ds(i * _EB, _EB)
            ee_row = jnp.sum(ea_ref[:, sl] * w_ae, axis=0)
            return jnp.maximum(m, jnp.max(ee_row))

        ee_max = jax.lax.fori_loop(0, _NB, ee_blk, jnp.float32(-jnp.inf))
        s_s_max = jnp.max(jnp.sum(h * a_s, axis=1))
        C = _lrelu(s_s_max + jnp.max(s_d) + ee_max)

        acc_ref[...] = jnp.zeros(acc_ref.shape, f32)

        def blk(i, carry, h=h, We=We, a_s=a_s, a_e=a_e, s_dw=s_dw, C=C):
            sl = pl.ds(i * _EB, _EB)
            src = ei_ref[0:1, sl]
            dst = ei_ref[1:2, sl]
            ea_b = ea_ref[:, sl]

            rows = jax.lax.broadcasted_iota(jnp.int32, (_N, _EB), 0)
            oh_src_t = (src == rows).astype(f32)
            oh_dst_t = (dst == rows).astype(f32)

            hs = jax.lax.dot_general(oh_src_t, h, _T_LHS,
                                     preferred_element_type=f32)
            sdg = jax.lax.dot_general(oh_dst_t, s_dw, _T_LHS,
                                      preferred_element_type=f32)[:, 0:1]
            he = jax.lax.dot_general(ea_b, We, _T_LHS,
                                     preferred_element_type=f32)

            e_s = jnp.sum(hs * a_s, axis=1, keepdims=True)
            ee = jnp.sum(he * a_e, axis=1, keepdims=True)
            e = _lrelu(e_s + sdg + ee)
            ex = jnp.exp(e - C)

            msg = jnp.concatenate(
                [ex * (hs + he), jnp.broadcast_to(ex, (_EB, 128))], axis=1)
            acc_ref[...] += jnp.dot(oh_dst_t, msg, preferred_element_type=f32)
            return carry

        jax.lax.fori_loop(0, _NB, blk, 0)

        acc = acc_ref[...]
        out = acc[:, 0:256] / (acc[:, 256:257] + 1e-16) + b_r[0:1, :]
        if li < 2:
            hin_ref[...] = _elu(out)
        else:
            hfin = out

    mean_p = jnp.mean(hfin, axis=0, keepdims=True)
    max_p = jnp.max(hfin, axis=0, keepdims=True)
    ctx = jnp.concatenate([mean_p, max_p], axis=1)
    q = jnp.dot(ctx, Wout_ref[...], preferred_element_type=f32) + bout_ref[0:1, :]
    q = jnp.where(jnp.isfinite(q), q, 0.0)
    out_ref[0] = jnp.broadcast_to(q, (8, 512))


def kernel(node_features, edge_features, action_mask, edge_index,
           ln_n_g, ln_n_b, ln_e_g, ln_e_b,
           W0, We0, as0, ad0, ae0, b0,
           W1, We1, as1, ad1, ae1, b1,
           W2, We2, as2, ad2, ae2, b2,
           W_out, b_out):
    f32 = _F32
    del action_mask

    rows8 = lambda v: jnp.tile(v.reshape(1, -1).astype(f32), (8, 1))
    cols128 = lambda v: jnp.tile(v.reshape(-1, 1).astype(f32), (1, 128))

    ef_t = jnp.transpose(edge_features.astype(f32), (0, 2, 1))
    ei_pad = jnp.concatenate([edge_index.astype(jnp.int32)] * 4, axis=0)

    args = (
        node_features.astype(f32), ef_t, ei_pad,
        rows8(ln_n_g), rows8(ln_n_b), cols128(ln_e_g), cols128(ln_e_b),
        W0.astype(f32), We0.astype(f32), rows8(as0), rows8(ad0), rows8(ae0), rows8(b0),
        W1.astype(f32), We1.astype(f32), rows8(as1), rows8(ad1), rows8(ae1), rows8(b1),
        W2.astype(f32), We2.astype(f32), rows8(as2), rows8(ad2), rows8(ae2), rows8(b2),
        W_out.astype(f32), rows8(b_out),
    )

    full = lambda a: pl.BlockSpec(a.shape, lambda b: (0,) * a.ndim)
    in_specs = [
        pl.BlockSpec((1, _N, node_features.shape[-1]), lambda b: (b, 0, 0)),
        pl.BlockSpec((1, 16, _E), lambda b: (b, 0, 0)),
        full(ei_pad),
    ] + [full(a) for a in args[3:]]

    out = pl.pallas_call(
        _gat_fwd_kernel,
        grid=(_B,),
        in_specs=in_specs,
        out_specs=pl.BlockSpec((1, 8, 512), lambda b: (b, 0, 0)),
        out_shape=jax.ShapeDtypeStruct((_B, 8, 512), f32),
        scratch_shapes=[
            pltpu.VMEM((16, _E), f32),
            pltpu.VMEM((_N, 256), f32),
            pltpu.VMEM((_N, 256), f32),
            pltpu.VMEM((_N, 384), f32),
        ],
    )(*args)
    return out[:, 0, :]


# speedup vs baseline: 4.9821x; 1.2205x over previous
import jax
import jax.numpy as jnp
from jax.experimental import pallas as pl
from jax.experimental.pallas import tpu as pltpu

_B, _N, _E = 8, 2048, 32768
_EB = 512
_NB = _E // _EB
_F32 = jnp.float32

_T_LHS = (((0,), (0,)), ((), ()))


def _lrelu(x):
    return jnp.where(x >= 0, x, 0.2 * x)


def _elu(x):
    return jnp.where(x > 0, x, jnp.exp(jnp.minimum(x, 0.0)) - 1.0)


def _gat_fwd_kernel(nf_ref, ef_ref, ei_ref,
                    ln_ng_ref, ln_nb_ref, ln_eg_ref, ln_eb_ref,
                    W0_ref, We0_ref, as0_ref, ad0_ref, ae0_ref, b0_ref,
                    W1_ref, We1_ref, as1_ref, ad1_ref, ae1_ref, b1_ref,
                    W2_ref, We2_ref, as2_ref, ad2_ref, ae2_ref, b2_ref,
                    Wout_ref, bout_ref,
                    out_ref,
                    ea_ref, hin_ref, hproj_ref, acc_ref):
    f32 = _F32

    x = nf_ref[0]
    mu = jnp.mean(x, axis=-1, keepdims=True)
    var = jnp.mean((x - mu) ** 2, axis=-1, keepdims=True)
    x = (x - mu) * jax.lax.rsqrt(var + 1e-5) * ln_ng_ref[0:1, :] + ln_nb_ref[0:1, :]

    g_e = ln_eg_ref[:, 0:1]
    b_e = ln_eb_ref[:, 0:1]

    def ln_blk(i, carry):
        sl = pl.ds(i * _EB, _EB)
        ea = ef_ref[0, :, sl]
        mu_e = jnp.mean(ea, axis=0, keepdims=True)
        var_e = jnp.mean((ea - mu_e) ** 2, axis=0, keepdims=True)
        ea_ref[:, sl] = (ea - mu_e) * jax.lax.rsqrt(var_e + 1e-5) * g_e + b_e
        return carry

    jax.lax.fori_loop(0, _NB, ln_blk, 0)

    layers = (
        (W0_ref, We0_ref, as0_ref, ad0_ref, ae0_ref, b0_ref),
        (W1_ref, We1_ref, as1_ref, ad1_ref, ae1_ref, b1_ref),
        (W2_ref, We2_ref, as2_ref, ad2_ref, ae2_ref, b2_ref),
    )

    hfin = None
    for li, (W_r, We_r, as_r, ad_r, ae_r, b_r) in enumerate(layers):
        W = W_r[...]
        We = We_r[...]
        a_s = as_r[0:1, :]
        a_d = ad_r[0:1, :]
        a_e = ae_r[0:1, :]

        h_prev = x if li == 0 else hin_ref[...]
        hproj_ref[...] = jnp.dot(h_prev, W, preferred_element_type=f32)
        h = hproj_ref[...]

        s_d = jnp.sum(h * a_d, axis=1, keepdims=True)
        s_dw = jnp.broadcast_to(s_d, (_N, 128))
        w_ae = jnp.sum(We * a_e, axis=1, keepdims=True)

        def ee_blk(i, m):
            sl = pl.ds(i * _EB, _EB)
            ee_row = jnp.sum(ea_ref[:, sl] * w_ae, axis=0)
            return jnp.maximum(m, jnp.max(ee_row))

        ee_max = jax.lax.fori_loop(0, _NB, ee_blk, jnp.float32(-jnp.inf))
        s_s_max = jnp.max(jnp.sum(h * a_s, axis=1))
        C = _lrelu(s_s_max + jnp.max(s_d) + ee_max)

        acc_ref[...] = jnp.zeros(acc_ref.shape, f32)

        def blk(i, carry, h=h, We=We, a_s=a_s, a_e=a_e, s_dw=s_dw, C=C):
            sl = pl.ds(i * _EB, _EB)
            src = ei_ref[0:1, sl]
            dst = ei_ref[1:2, sl]
            ea_b = ea_ref[:, sl]

            rows = jax.lax.broadcasted_iota(jnp.int32, (_N, _EB), 0)
            oh_src_t = (src == rows).astype(f32)
            oh_dst_t = (dst == rows).astype(f32)

            hs = jax.lax.dot_general(oh_src_t, h, _T_LHS,
                                     preferred_element_type=f32)
            sdg = jax.lax.dot_general(oh_dst_t, s_dw, _T_LHS,
                                      preferred_element_type=f32)[:, 0:1]
            he = jax.lax.dot_general(ea_b, We, _T_LHS,
                                     preferred_element_type=f32)

            e_s = jnp.sum(hs * a_s, axis=1, keepdims=True)
            ee = jnp.sum(he * a_e, axis=1, keepdims=True)
            e = _lrelu(e_s + sdg + ee)
            ex = jnp.exp(e - C)

            msg = jnp.concatenate(
                [ex * (hs + he), jnp.broadcast_to(ex, (_EB, 128))], axis=1)
            acc_ref[...] += jnp.dot(oh_dst_t, msg, preferred_element_type=f32)
            return carry

        jax.lax.fori_loop(0, _NB, blk, 0)

        acc = acc_ref[...]
        out = acc[:, 0:256] / (acc[:, 256:257] + 1e-16) + b_r[0:1, :]
        if li < 2:
            hin_ref[...] = _elu(out)
        else:
            hfin = out

    mean_p = jnp.mean(hfin, axis=0, keepdims=True)
    max_p = jnp.max(hfin, axis=0, keepdims=True)
    ctx = jnp.concatenate([mean_p, max_p], axis=1)
    q = jnp.dot(ctx, Wout_ref[...], preferred_element_type=f32) + bout_ref[0:1, :]
    q = jnp.where(jnp.isfinite(q), q, 0.0)
    out_ref[0] = jnp.broadcast_to(q, (8, 512))


def kernel(node_features, edge_features, action_mask, edge_index,
           ln_n_g, ln_n_b, ln_e_g, ln_e_b,
           W0, We0, as0, ad0, ae0, b0,
           W1, We1, as1, ad1, ae1, b1,
           W2, We2, as2, ad2, ae2, b2,
           W_out, b_out):
    f32 = _F32
    del action_mask

    rows8 = lambda v: jnp.tile(v.reshape(1, -1).astype(f32), (8, 1))
    cols128 = lambda v: jnp.tile(v.reshape(-1, 1).astype(f32), (1, 128))

    ef_t = jnp.transpose(edge_features.astype(f32), (0, 2, 1))
    ei_pad = jnp.concatenate([edge_index.astype(jnp.int32)] * 4, axis=0)

    args = (
        node_features.astype(f32), ef_t, ei_pad,
        rows8(ln_n_g), rows8(ln_n_b), cols128(ln_e_g), cols128(ln_e_b),
        W0.astype(f32), We0.astype(f32), rows8(as0), rows8(ad0), rows8(ae0), rows8(b0),
        W1.astype(f32), We1.astype(f32), rows8(as1), rows8(ad1), rows8(ae1), rows8(b1),
        W2.astype(f32), We2.astype(f32), rows8(as2), rows8(ad2), rows8(ae2), rows8(b2),
        W_out.astype(f32), rows8(b_out),
    )

    full = lambda a: pl.BlockSpec(a.shape, lambda b: (0,) * a.ndim)
    in_specs = [
        pl.BlockSpec((1, _N, node_features.shape[-1]), lambda b: (b, 0, 0)),
        pl.BlockSpec((1, 16, _E), lambda b: (b, 0, 0)),
        full(ei_pad),
    ] + [full(a) for a in args[3:]]

    out = pl.pallas_call(
        _gat_fwd_kernel,
        grid=(_B,),
        in_specs=in_specs,
        out_specs=pl.BlockSpec((1, 8, 512), lambda b: (b, 0, 0)),
        out_shape=jax.ShapeDtypeStruct((_B, 8, 512), f32),
        scratch_shapes=[
            pltpu.VMEM((16, _E), f32),
            pltpu.VMEM((_N, 256), f32),
            pltpu.VMEM((_N, 256), f32),
            pltpu.VMEM((_N, 384), f32),
        ],
    )(*args)
    return out[:, 0, :]
